# baseline (speedup 1.0000x reference)
import functools

import jax
import jax.numpy as jnp
from jax import lax
from jax.experimental import pallas as pl
from jax.experimental.pallas import tpu as pltpu
from jax.experimental.pallas import tpu_sc as plsc

N = 10000
E = 160000
DIN = 256
DH = 256
DOUT = 128
NE = 8

NSC = 2
NTILES = 16
CHUNK = 80
DIST = 5
NB = 2 * DIST
EDGES_PER_TILE = E // NTILES
CHUNKS_PER_TILE = EDGES_PER_TILE // CHUNK
ROWS_MAIN = 624
TAIL0 = NTILES * ROWS_MAIN
TAILN = N - TAIL0

_f32 = jnp.float32


@functools.lru_cache(maxsize=None)
def _build_segsum(qw: int, with_counts: bool, single: bool = False,
                  interpret: bool = False):
    mesh = plsc.VectorSubcoreMesh(core_axis_name="c", subcore_axis_name="s",
                                  num_cores=NSC, num_subcores=NTILES)
    n_arr = 1 if single else 2
    mul = 128 // qw
    out_type = [jax.ShapeDtypeStruct((N, 128), _f32) for _ in range(n_arr)]
    if with_counts:
        out_type.append(jax.ShapeDtypeStruct((N, 16), _f32))
    scratch = [
        pltpu.VMEM((CHUNKS_PER_TILE, CHUNK), jnp.int32),
        pltpu.VMEM((CHUNKS_PER_TILE, CHUNK), jnp.int32),
        pltpu.VMEM((NB, CHUNK, qw), _f32),
        pltpu.VMEM((CHUNK, 16), _f32),
        pltpu.VMEM_SHARED((N, qw), _f32),
    ]
    if with_counts:
        scratch.append(pltpu.VMEM_SHARED((N, 16), _f32))
    scratch += [pltpu.SemaphoreType.DMA] * (2 * NB + 1)

    def body(*args):
        xs = list(args[:n_arr])
        src_hbm, dst_hbm, z_hbm, z16_hbm = args[n_arr:n_arr + 4]
        rest = args[n_arr + 4:]
        outs = list(rest[:n_arr])
        rest = rest[n_arr:]
        if with_counts:
            cnt_out = rest[0]
            rest = rest[1:]
        src_v, dst_v, rows_v, ones_v, acc = rest[:5]
        rest = rest[5:]
        if with_counts:
            cnt_acc = rest[0]
            rest = rest[1:]
        gsem = rest[:NB]
        ssem = rest[NB:2 * NB]
        csem = rest[2 * NB]
        cid = lax.axis_index("c")
        tid = lax.axis_index("s")

        row0 = tid * ROWS_MAIN
        last = tid == NTILES - 1

        def sliced_copy(src, dst, dst_p=None):
            def sl(ref, p, r0, nr):
                if p is None:
                    return ref.at[pl.ds(r0, nr)]
                return ref.at[pl.ds(r0, nr), pl.ds(qw * p, qw)]

            pltpu.sync_copy(sl(src, None, row0, ROWS_MAIN),
                            sl(dst, dst_p, row0, ROWS_MAIN))

            @pl.when(last)
            def _():
                pltpu.sync_copy(sl(src, None, TAIL0, TAILN),
                                sl(dst, dst_p, TAIL0, TAILN))

        def start_gather(x_hbm, j, b):
            return pltpu.async_copy(x_hbm.at[src_v.at[j]], rows_v.at[b],
                                    gsem[b])

        def wait_gather(x_hbm, j, b):
            pltpu.make_async_copy(x_hbm.at[src_v.at[j]], rows_v.at[b],
                                  gsem[b]).wait()

        def transform_src(mul_now, add_now):
            addv = jnp.full((16,), add_now, jnp.int32)

            def trow(r, _):
                for k in range(CHUNK // 16):
                    v = src_v[r, pl.ds(16 * k, 16)]
                    src_v[r, pl.ds(16 * k, 16)] = v * mul_now + addv
                return 0

            lax.fori_loop(0, CHUNKS_PER_TILE, trow, 0)

        def wait_scat(j, b):
            pltpu.make_async_copy(rows_v.at[b], acc.at[dst_v.at[j]],
                                  ssem[b]).wait()

        def wait_cnt(j, cnt_acc):
            pltpu.make_async_copy(ones_v, cnt_acc.at[dst_v.at[j]],
                                  csem).wait()

        for c in range(NSC):
            @pl.when(cid == c)
            def _(c=c):
                pltpu.sync_copy(src_hbm.at[tid], src_v)
                pltpu.sync_copy(dst_hbm.at[tid], dst_v)
                if with_counts and c == 0:
                    for r in range(CHUNK):
                        ones_v[r] = jnp.full((16,), 1.0, _f32)
                for p in range(2):
                    do_cnt = with_counts and c == 0 and p == 0
                    if single:
                        x_hbm, out, slot = xs[0], outs[0], 2 * c + p
                    else:
                        x_hbm, out, slot = xs[c], outs[c], p
                    if p == 0:
                        transform_src(mul, 2 * c if single else 0)
                    else:
                        transform_src(1, 1)
                    sliced_copy(z_hbm, acc)
                    if do_cnt:
                        sliced_copy(z16_hbm, cnt_acc)
                    plsc.subcore_barrier()
                    for b in range(DIST):
                        start_gather(x_hbm, b, b)

                    def process(j, b, prefetch, x_hbm=x_hbm, do_cnt=do_cnt):
                        wait_gather(x_hbm, j, b)
                        pltpu.async_copy(rows_v.at[b], acc.at[dst_v.at[j]],
                                         ssem[b], add=True)
                        if do_cnt:
                            pltpu.async_copy(ones_v, cnt_acc.at[dst_v.at[j]],
                                             csem, add=True)

                            @pl.when(j >= DIST)
                            def _():
                                wait_cnt(j - DIST, cnt_acc)
                        if prefetch:
                            pj = j + DIST
                            pb = (b + DIST) % NB

                            @pl.when(j >= DIST)
                            def _():
                                wait_scat(j - DIST, pb)

                            start_gather(x_hbm, pj, pb)

                    def step(i, _):
                        j0 = i * NB
                        for b in range(NB):
                            process(j0 + b, b, True)
                        return 0

                    n_main = (CHUNKS_PER_TILE - DIST) // NB
                    lax.fori_loop(0, n_main, step, 0)
                    for b in range(DIST):
                        process(jnp.int32(n_main * NB + b), b, False)
                    for k in range(NB):
                        j = CHUNKS_PER_TILE - NB + k
                        wait_scat(j, j % NB)
                    if do_cnt:
                        for k in range(DIST):
                            wait_cnt(CHUNKS_PER_TILE - DIST + k, cnt_acc)
                    plsc.subcore_barrier()
                    sliced_copy(acc, out, dst_p=slot)
                    if do_cnt:
                        sliced_copy(cnt_acc, cnt_out)

    return pl.kernel(body, out_type=tuple(out_type), mesh=mesh,
                     scratch_types=tuple(scratch), interpret=interpret,
                     compiler_params=pltpu.CompilerParams(
                         use_tc_tiling_on_sc=False))


def _segsum(xhs, src3, dst3, qw, with_counts, single, z16):
    zq = jnp.zeros((N, qw), _f32)
    n_arr = len(xhs)
    flat = tuple(a.reshape(-1, qw) for a in xhs)
    res = _build_segsum(qw, with_counts, single)(*flat, src3, dst3, zq, z16)
    outs = list(res[:n_arr])
    if with_counts:
        outs.append(res[n_arr])
    return outs


TB = 1000


def _root_body(xa_ref, xb_ref, Wr_ref, out_ref):
    x = jnp.concatenate([xa_ref[...], xb_ref[...]], axis=1)
    out_ref[...] = jnp.dot(x, Wr_ref[...], preferred_element_type=_f32)


@functools.lru_cache(maxsize=None)
def _make_root(dout: int, interpret: bool = False):
    return pl.pallas_call(
        _root_body, grid=(N // TB,),
        in_specs=[pl.BlockSpec((TB, 128), lambda i: (i, 0)),
                  pl.BlockSpec((TB, 128), lambda i: (i, 0)),
                  pl.BlockSpec((DH, dout), lambda i: (0, 0))],
        out_specs=pl.BlockSpec((TB, dout), lambda i: (i, 0)),
        out_shape=jax.ShapeDtypeStruct((N, dout), _f32),
        interpret=interpret)


def _moe_body(sh_refs, cnt_ref, root_ref, Wg_ref, bg_ref, W_ref, b_ref,
              Wl2_ref, outh_refs, y2_ref, gs_ref):
    i = pl.program_id(0)
    s = jnp.concatenate([r[...] for r in sh_refs], axis=1)
    cnt = jnp.maximum(cnt_ref[...][:, 0:1], 1.0)
    h = s / cnt
    logits = jnp.dot(h, Wg_ref[...], preferred_element_type=_f32) + bg_ref[...]
    m = jnp.max(logits, axis=1, keepdims=True)
    eg = jnp.exp(logits - m)
    g = eg / jnp.sum(eg, axis=1, keepdims=True)
    gm = jnp.mean(g, axis=1, keepdims=True)
    stds = jnp.sqrt(jnp.sum((g - gm) ** 2, axis=1, keepdims=True) / (NE - 1))

    @pl.when(i == 0)
    def _():
        gs_ref[...] = jnp.zeros((1, 1), _f32)

    gs_ref[...] = gs_ref[...] + jnp.sum(stds).reshape(1, 1)

    acc = root_ref[...]
    found = jnp.zeros((TB, 1), jnp.bool_)
    for e in range(NE):
        is_e = jnp.logical_and(logits[:, e:e + 1] == m,
                               jnp.logical_not(found))
        found = jnp.logical_or(found, is_e)
        mask = is_e.astype(_f32)
        acc += jnp.dot(h * mask, W_ref[e],
                       preferred_element_type=_f32) + mask * b_ref[e:e + 1, :]
    xn = jnp.maximum(acc, 0.0)
    outh_refs[0][...] = xn[:, :128]
    outh_refs[1][...] = xn[:, 128:]
    if y2_ref is not None:
        y2_ref[...] = jnp.dot(xn, Wl2_ref[...], preferred_element_type=_f32)


@functools.lru_cache(maxsize=None)
def _make_moe_layer(with_y2: bool, interpret: bool = False):
    def body(*refs):
        sh = refs[0:2]
        cnt = refs[2]
        root = refs[3]
        Wg, bg, W, b = refs[4:8]
        k = 8
        Wl2 = refs[k] if with_y2 else None
        k += 1 if with_y2 else 0
        outh = refs[k:k + 2]
        k += 2
        y2 = refs[k] if with_y2 else None
        k += 1 if with_y2 else 0
        gs = refs[k]
        _moe_body(sh, cnt, root, Wg, bg, W, b, Wl2, outh, y2, gs)

    grid = (N // TB,)
    tok = lambda w: pl.BlockSpec((TB, w), lambda i: (i, 0))
    full = lambda *shape: pl.BlockSpec(shape, lambda i: tuple(0 for _ in shape))
    in_specs = [tok(128)] * 2 + [tok(16)] + [tok(DH)] + [
        full(DH, NE), full(1, NE), full(NE, DH, DH), full(NE, DH)]
    out_shapes = [jax.ShapeDtypeStruct((N, 128), _f32) for _ in range(2)]
    out_specs = [tok(128)] * 2
    if with_y2:
        in_specs.append(full(DH, DOUT))
        out_shapes.append(jax.ShapeDtypeStruct((N, DOUT), _f32))
        out_specs.append(tok(DOUT))
    out_shapes.append(jax.ShapeDtypeStruct((1, 1), _f32))
    out_specs.append(pl.BlockSpec((1, 1), lambda i: (0, 0)))
    return pl.pallas_call(
        body, grid=grid, in_specs=in_specs, out_specs=out_specs,
        out_shape=tuple(out_shapes), interpret=interpret)


def _final_body(s2_ref, cnt_ref, root_ref, bl2_ref, out_ref):
    cnt = jnp.maximum(cnt_ref[...][:, 0:1], 1.0)
    out_ref[...] = s2_ref[...] / cnt + bl2_ref[...] + root_ref[...]


@functools.lru_cache(maxsize=None)
def _make_final(interpret: bool = False):
    return pl.pallas_call(
        _final_body, grid=(N // TB,),
        in_specs=[pl.BlockSpec((TB, 128), lambda i: (i, 0)),
                  pl.BlockSpec((TB, 16), lambda i: (i, 0)),
                  pl.BlockSpec((TB, DOUT), lambda i: (i, 0)),
                  pl.BlockSpec((1, DOUT), lambda i: (0, 0))],
        out_specs=pl.BlockSpec((TB, DOUT), lambda i: (i, 0)),
        out_shape=jax.ShapeDtypeStruct((N, DOUT), _f32),
        interpret=interpret)


def kernel(x, edge_index, Wg0, bg0, W0, b0, Wr0, Wg1, bg1, W1, b1, Wr1,
           Wl2, bl2, Wr2):
    xh = (x[:, :128], x[:, 128:])
    src3 = edge_index[0].reshape(NTILES, CHUNKS_PER_TILE, CHUNK)
    dst3 = edge_index[1].reshape(NTILES, CHUNKS_PER_TILE, CHUNK)
    z16 = jnp.zeros((N, 16), _f32)

    root0 = _make_root(DH)(*xh, Wr0)
    *s0h, cnt16 = _segsum(xh, src3, dst3, 64, True, False, z16)
    r0 = _make_moe_layer(False)(*s0h, cnt16, root0, Wg0, bg0.reshape(1, NE),
                                W0, b0)
    x1h, gs0 = r0[:2], r0[2]
    root1 = _make_root(DH)(*x1h, Wr1)
    s1h = _segsum(x1h, src3, dst3, 64, False, False, z16)
    r1 = _make_moe_layer(True)(*s1h, cnt16, root1, Wg1, bg1.reshape(1, NE),
                               W1, b1, Wl2)
    x2h, y2, gs1 = r1[:2], r1[2], r1[3]
    root2 = _make_root(DOUT)(*x2h, Wr2)
    (s2,) = _segsum((y2,), src3, dst3, 32, False, True, z16)
    out = _make_final()(s2, cnt16, root2, bl2.reshape(1, DOUT))
    gstd = (gs0[0, 0] + gs1[0, 0]) / (2.0 * N)
    return out, gstd


# speedup vs baseline: 1.0484x; 1.0484x over previous
import functools

import jax
import jax.numpy as jnp
from jax import lax
from jax.experimental import pallas as pl
from jax.experimental.pallas import tpu as pltpu
from jax.experimental.pallas import tpu_sc as plsc

N = 10000
E = 160000
DIN = 256
DH = 256
DOUT = 128
NE = 8

NSC = 2
NTILES = 16
CHUNK = 80
DIST = 5
NB = 2 * DIST
EDGES_PER_TILE = E // NTILES
CHUNKS_PER_TILE = EDGES_PER_TILE // CHUNK
ROWS_MAIN = 624
TAIL0 = NTILES * ROWS_MAIN
TAILN = N - TAIL0

_f32 = jnp.float32


@functools.lru_cache(maxsize=None)
def _build_segsum(qw: int, with_counts: bool, single: bool = False,
                  interpret: bool = False):
    mesh = plsc.VectorSubcoreMesh(core_axis_name="c", subcore_axis_name="s",
                                  num_cores=NSC, num_subcores=NTILES)
    n_arr = 1 if single else 2
    mul = 128 // qw
    out_type = [jax.ShapeDtypeStruct((N, 128), _f32) for _ in range(n_arr)]
    if with_counts:
        out_type.append(jax.ShapeDtypeStruct((N, 16), _f32))
    scratch = [
        pltpu.VMEM((CHUNKS_PER_TILE, CHUNK), jnp.int32),
        pltpu.VMEM((CHUNKS_PER_TILE, CHUNK), jnp.int32),
        pltpu.VMEM((NB, CHUNK, qw), _f32),
        pltpu.VMEM((CHUNK, 16), _f32),
        pltpu.VMEM_SHARED((N, qw), _f32),
    ]
    if with_counts:
        scratch.append(pltpu.VMEM_SHARED((N, 16), _f32))
    scratch += [pltpu.SemaphoreType.DMA] * NB

    def body(*args):
        xs = list(args[:n_arr])
        src_hbm, dst_hbm, z_hbm, z16_hbm = args[n_arr:n_arr + 4]
        rest = args[n_arr + 4:]
        outs = list(rest[:n_arr])
        rest = rest[n_arr:]
        if with_counts:
            cnt_out = rest[0]
            rest = rest[1:]
        src_v, dst_v, rows_v, ones_v, acc = rest[:5]
        rest = rest[5:]
        if with_counts:
            cnt_acc = rest[0]
            rest = rest[1:]
        gsem = rest[:NB]
        cid = lax.axis_index("c")
        tid = lax.axis_index("s")

        row0 = tid * ROWS_MAIN
        last = tid == NTILES - 1

        def sliced_copy(src, dst, dst_p=None):
            def sl(ref, p, r0, nr):
                if p is None:
                    return ref.at[pl.ds(r0, nr)]
                return ref.at[pl.ds(r0, nr), pl.ds(qw * p, qw)]

            pltpu.sync_copy(sl(src, None, row0, ROWS_MAIN),
                            sl(dst, dst_p, row0, ROWS_MAIN))

            @pl.when(last)
            def _():
                pltpu.sync_copy(sl(src, None, TAIL0, TAILN),
                                sl(dst, dst_p, TAIL0, TAILN))

        def start_gather(x_hbm, j, b):
            return pltpu.async_copy(x_hbm.at[src_v.at[j]], rows_v.at[b],
                                    gsem[b])

        def wait_gather(x_hbm, j, b):
            pltpu.make_async_copy(x_hbm.at[src_v.at[j]], rows_v.at[b],
                                  gsem[b]).wait()

        def transform_src(mul_now, add_now):
            addv = jnp.full((16,), add_now, jnp.int32)

            def trow(r, _):
                for k in range(CHUNK // 16):
                    v = src_v[r, pl.ds(16 * k, 16)]
                    src_v[r, pl.ds(16 * k, 16)] = v * mul_now + addv
                return 0

            lax.fori_loop(0, CHUNKS_PER_TILE, trow, 0)

        for c in range(NSC):
            @pl.when(cid == c)
            def _(c=c):
                pltpu.sync_copy(src_hbm.at[tid], src_v)
                pltpu.sync_copy(dst_hbm.at[tid], dst_v)
                if with_counts and c == 0:
                    for r in range(CHUNK):
                        ones_v[r] = jnp.full((16,), 1.0, _f32)
                for p in range(2):
                    do_cnt = with_counts and c == 0 and p == 0
                    if single:
                        x_hbm, out, slot = xs[0], outs[0], 2 * c + p
                    else:
                        x_hbm, out, slot = xs[c], outs[c], p
                    if p == 0:
                        transform_src(mul, 2 * c if single else 0)
                    else:
                        transform_src(1, 1)
                    sliced_copy(z_hbm, acc)
                    if do_cnt:
                        sliced_copy(z16_hbm, cnt_acc)
                    plsc.subcore_barrier()
                    for b in range(DIST):
                        start_gather(x_hbm, b, b)

                    def process(j, b, prefetch, x_hbm=x_hbm, do_cnt=do_cnt):
                        wait_gather(x_hbm, j, b)
                        pltpu.sync_copy(rows_v.at[b], acc.at[dst_v.at[j]],
                                        add=True)
                        if do_cnt:
                            pltpu.sync_copy(ones_v, cnt_acc.at[dst_v.at[j]],
                                            add=True)
                        if prefetch:
                            start_gather(x_hbm, j + DIST, (b + DIST) % NB)

                    def step(i, _):
                        j0 = i * NB
                        for b in range(NB):
                            process(j0 + b, b, True)
                        return 0

                    n_main = (CHUNKS_PER_TILE - DIST) // NB
                    lax.fori_loop(0, n_main, step, 0)
                    for b in range(DIST):
                        process(jnp.int32(n_main * NB + b), b, False)
                    plsc.subcore_barrier()
                    sliced_copy(acc, out, dst_p=slot)
                    if do_cnt:
                        sliced_copy(cnt_acc, cnt_out)

    return pl.kernel(body, out_type=tuple(out_type), mesh=mesh,
                     scratch_types=tuple(scratch), interpret=interpret,
                     compiler_params=pltpu.CompilerParams(
                         use_tc_tiling_on_sc=False))


def _segsum(xhs, src3, dst3, qw, with_counts, single, z16):
    zq = jnp.zeros((N, qw), _f32)
    n_arr = len(xhs)
    flat = tuple(a.reshape(-1, qw) for a in xhs)
    res = _build_segsum(qw, with_counts, single)(*flat, src3, dst3, zq, z16)
    outs = list(res[:n_arr])
    if with_counts:
        outs.append(res[n_arr])
    return outs


TB = 1000


def _moe_body(sh_refs, cnt_ref, xh_refs, Wg_ref, bg_ref, W_ref, b_ref,
              Wr_ref, Wl2_ref, outh_refs, y2_ref, gs_ref):
    i = pl.program_id(0)
    s = jnp.concatenate([r[...] for r in sh_refs], axis=1)
    cnt = jnp.maximum(cnt_ref[...][:, 0:1], 1.0)
    h = s / cnt
    logits = jnp.dot(h, Wg_ref[...], preferred_element_type=_f32) + bg_ref[...]
    m = jnp.max(logits, axis=1, keepdims=True)
    eg = jnp.exp(logits - m)
    g = eg / jnp.sum(eg, axis=1, keepdims=True)
    gm = jnp.mean(g, axis=1, keepdims=True)
    stds = jnp.sqrt(jnp.sum((g - gm) ** 2, axis=1, keepdims=True) / (NE - 1))

    @pl.when(i == 0)
    def _():
        gs_ref[...] = jnp.zeros((1, 1), _f32)

    gs_ref[...] = gs_ref[...] + jnp.sum(stds).reshape(1, 1)

    x = jnp.concatenate([r[...] for r in xh_refs], axis=1)
    acc = jnp.dot(x, Wr_ref[...], preferred_element_type=_f32)
    found = jnp.zeros((TB, 1), jnp.bool_)
    for e in range(NE):
        is_e = jnp.logical_and(logits[:, e:e + 1] == m,
                               jnp.logical_not(found))
        found = jnp.logical_or(found, is_e)
        mask = is_e.astype(_f32)
        acc += jnp.dot(h * mask, W_ref[e],
                       preferred_element_type=_f32) + mask * b_ref[e:e + 1, :]
    xn = jnp.maximum(acc, 0.0)
    outh_refs[0][...] = xn[:, :128]
    outh_refs[1][...] = xn[:, 128:]
    if y2_ref is not None:
        y2_ref[...] = jnp.dot(xn, Wl2_ref[...], preferred_element_type=_f32)


@functools.lru_cache(maxsize=None)
def _make_moe_layer(with_y2: bool, interpret: bool = False):
    def body(*refs):
        sh = refs[0:2]
        cnt = refs[2]
        xh = refs[3:5]
        Wg, bg, W, b, Wr = refs[5:10]
        k = 10
        Wl2 = refs[k] if with_y2 else None
        k += 1 if with_y2 else 0
        outh = refs[k:k + 2]
        k += 2
        y2 = refs[k] if with_y2 else None
        k += 1 if with_y2 else 0
        gs = refs[k]
        _moe_body(sh, cnt, xh, Wg, bg, W, b, Wr, Wl2, outh, y2, gs)

    grid = (N // TB,)
    tok = lambda w: pl.BlockSpec((TB, w), lambda i: (i, 0))
    full = lambda *shape: pl.BlockSpec(shape, lambda i: tuple(0 for _ in shape))
    in_specs = [tok(128)] * 2 + [tok(16)] + [tok(128)] * 2 + [
        full(DH, NE), full(1, NE), full(NE, DH, DH), full(NE, DH),
        full(DH, DH)]
    out_shapes = [jax.ShapeDtypeStruct((N, 128), _f32) for _ in range(2)]
    out_specs = [tok(128)] * 2
    if with_y2:
        in_specs.append(full(DH, DOUT))
        out_shapes.append(jax.ShapeDtypeStruct((N, DOUT), _f32))
        out_specs.append(tok(DOUT))
    out_shapes.append(jax.ShapeDtypeStruct((1, 1), _f32))
    out_specs.append(pl.BlockSpec((1, 1), lambda i: (0, 0)))
    return pl.pallas_call(
        body, grid=grid, in_specs=in_specs, out_specs=out_specs,
        out_shape=tuple(out_shapes), interpret=interpret)


def _final_body(s2_ref, cnt_ref, xa_ref, xb_ref, Wr2_ref, bl2_ref, out_ref):
    cnt = jnp.maximum(cnt_ref[...][:, 0:1], 1.0)
    x2 = jnp.concatenate([xa_ref[...], xb_ref[...]], axis=1)
    out_ref[...] = (s2_ref[...] / cnt + bl2_ref[...]
                    + jnp.dot(x2, Wr2_ref[...], preferred_element_type=_f32))


@functools.lru_cache(maxsize=None)
def _make_final(interpret: bool = False):
    return pl.pallas_call(
        _final_body, grid=(N // TB,),
        in_specs=[pl.BlockSpec((TB, 128), lambda i: (i, 0)),
                  pl.BlockSpec((TB, 16), lambda i: (i, 0)),
                  pl.BlockSpec((TB, 128), lambda i: (i, 0)),
                  pl.BlockSpec((TB, 128), lambda i: (i, 0)),
                  pl.BlockSpec((DH, DOUT), lambda i: (0, 0)),
                  pl.BlockSpec((1, DOUT), lambda i: (0, 0))],
        out_specs=pl.BlockSpec((TB, DOUT), lambda i: (i, 0)),
        out_shape=jax.ShapeDtypeStruct((N, DOUT), _f32),
        interpret=interpret)


def kernel(x, edge_index, Wg0, bg0, W0, b0, Wr0, Wg1, bg1, W1, b1, Wr1,
           Wl2, bl2, Wr2):
    xh = (x[:, :128], x[:, 128:])
    src3 = edge_index[0].reshape(NTILES, CHUNKS_PER_TILE, CHUNK)
    dst3 = edge_index[1].reshape(NTILES, CHUNKS_PER_TILE, CHUNK)
    z16 = jnp.zeros((N, 16), _f32)

    *s0h, cnt16 = _segsum(xh, src3, dst3, 64, True, False, z16)
    r0 = _make_moe_layer(False)(*s0h, cnt16, *xh, Wg0, bg0.reshape(1, NE),
                                W0, b0, Wr0)
    x1h, gs0 = r0[:2], r0[2]
    s1h = _segsum(x1h, src3, dst3, 64, False, False, z16)
    r1 = _make_moe_layer(True)(*s1h, cnt16, *x1h, Wg1, bg1.reshape(1, NE),
                               W1, b1, Wr1, Wl2)
    x2h, y2, gs1 = r1[:2], r1[2], r1[3]
    (s2,) = _segsum((y2,), src3, dst3, 32, False, True, z16)
    out = _make_final()(s2, cnt16, *x2h, Wr2, bl2.reshape(1, DOUT))
    gstd = (gs0[0, 0] + gs1[0, 0]) / (2.0 * N)
    return out, gstd


# speedup vs baseline: 1.0533x; 1.0046x over previous
import functools

import jax
import jax.numpy as jnp
from jax import lax
from jax.experimental import pallas as pl
from jax.experimental.pallas import tpu as pltpu
from jax.experimental.pallas import tpu_sc as plsc

N = 10000
E = 160000
DIN = 256
DH = 256
DOUT = 128
NE = 8

NSC = 2
NTILES = 16
CHUNK = 80
DIST = 5
NB = 2 * DIST
EDGES_PER_TILE = E // NTILES
CHUNKS_PER_TILE = EDGES_PER_TILE // CHUNK
ROWS_MAIN = 624
TAIL0 = NTILES * ROWS_MAIN
TAILN = N - TAIL0

_f32 = jnp.float32


@functools.lru_cache(maxsize=None)
def _build_segsum(qw: int, with_counts: bool, single: bool = False,
                  interpret: bool = False):
    mesh = plsc.VectorSubcoreMesh(core_axis_name="c", subcore_axis_name="s",
                                  num_cores=NSC, num_subcores=NTILES)
    n_arr = 1 if single else 2
    mul = 128 // qw
    out_type = [jax.ShapeDtypeStruct((N, 128), _f32) for _ in range(n_arr)]
    if with_counts:
        out_type.append(jax.ShapeDtypeStruct((N, 16), _f32))
    scratch = [
        pltpu.VMEM((CHUNKS_PER_TILE, CHUNK), jnp.int32),
        pltpu.VMEM((CHUNKS_PER_TILE, CHUNK), jnp.int32),
        pltpu.VMEM((NB, CHUNK, qw), _f32),
        pltpu.VMEM((CHUNK, 16), _f32),
        pltpu.VMEM_SHARED((N, qw), _f32),
    ]
    if with_counts:
        scratch.append(pltpu.VMEM_SHARED((N, 16), _f32))
    scratch += [pltpu.SemaphoreType.DMA] * NB

    def body(*args):
        xs = list(args[:n_arr])
        src_hbm, dst_hbm, z_hbm, z16_hbm = args[n_arr:n_arr + 4]
        rest = args[n_arr + 4:]
        outs = list(rest[:n_arr])
        rest = rest[n_arr:]
        if with_counts:
            cnt_out = rest[0]
            rest = rest[1:]
        src_v, dst_v, rows_v, ones_v, acc = rest[:5]
        rest = rest[5:]
        if with_counts:
            cnt_acc = rest[0]
            rest = rest[1:]
        gsem = rest[:NB]
        cid = lax.axis_index("c")
        tid = lax.axis_index("s")

        row0 = tid * ROWS_MAIN
        last = tid == NTILES - 1

        def sliced_copy(src, dst, dst_p=None):
            def sl(ref, p, r0, nr):
                if p is None:
                    return ref.at[pl.ds(r0, nr)]
                return ref.at[pl.ds(r0, nr), pl.ds(qw * p, qw)]

            pltpu.sync_copy(sl(src, None, row0, ROWS_MAIN),
                            sl(dst, dst_p, row0, ROWS_MAIN))

            @pl.when(last)
            def _():
                pltpu.sync_copy(sl(src, None, TAIL0, TAILN),
                                sl(dst, dst_p, TAIL0, TAILN))

        def start_gather(x_hbm, j, b):
            return pltpu.async_copy(x_hbm.at[src_v.at[j]], rows_v.at[b],
                                    gsem[b])

        def wait_gather(x_hbm, j, b):
            pltpu.make_async_copy(x_hbm.at[src_v.at[j]], rows_v.at[b],
                                  gsem[b]).wait()

        def transform_src(mul_now, add_now):
            addv = jnp.full((16,), add_now, jnp.int32)

            def trow(r, _):
                for k in range(CHUNK // 16):
                    v = src_v[r, pl.ds(16 * k, 16)]
                    src_v[r, pl.ds(16 * k, 16)] = v * mul_now + addv
                return 0

            lax.fori_loop(0, CHUNKS_PER_TILE, trow, 0)

        for c in range(NSC):
            @pl.when(cid == c)
            def _(c=c):
                pltpu.sync_copy(src_hbm.at[tid], src_v)
                pltpu.sync_copy(dst_hbm.at[tid], dst_v)
                if with_counts and c == 0:
                    for r in range(CHUNK):
                        ones_v[r] = jnp.full((16,), 1.0, _f32)
                for p in range(2):
                    do_cnt = with_counts and c == 0 and p == 0
                    if single:
                        x_hbm, out, slot = xs[0], outs[0], 2 * c + p
                    else:
                        x_hbm, out, slot = xs[c], outs[c], p
                    if p == 0:
                        transform_src(mul, 2 * c if single else 0)
                    else:
                        transform_src(1, 1)
                    sliced_copy(z_hbm, acc)
                    if do_cnt:
                        sliced_copy(z16_hbm, cnt_acc)
                    plsc.subcore_barrier()
                    for b in range(DIST):
                        start_gather(x_hbm, b, b)

                    def process(j, b, prefetch, x_hbm=x_hbm, do_cnt=do_cnt):
                        wait_gather(x_hbm, j, b)
                        pltpu.sync_copy(rows_v.at[b], acc.at[dst_v.at[j]],
                                        add=True)
                        if do_cnt:
                            pltpu.sync_copy(ones_v, cnt_acc.at[dst_v.at[j]],
                                            add=True)
                        if prefetch:
                            start_gather(x_hbm, j + DIST, (b + DIST) % NB)

                    def step(i, _):
                        j0 = i * NB
                        for b in range(NB):
                            process(j0 + b, b, True)
                        return 0

                    n_main = (CHUNKS_PER_TILE - DIST) // NB
                    lax.fori_loop(0, n_main, step, 0)
                    for b in range(DIST):
                        process(jnp.int32(n_main * NB + b), b, False)
                    plsc.subcore_barrier()
                    sliced_copy(acc, out, dst_p=slot)
                    if do_cnt:
                        sliced_copy(cnt_acc, cnt_out)

    return pl.kernel(body, out_type=tuple(out_type), mesh=mesh,
                     scratch_types=tuple(scratch), interpret=interpret,
                     compiler_params=pltpu.CompilerParams(
                         use_tc_tiling_on_sc=False))


def _segsum(xhs, src3, dst3, qw, with_counts, single, z16):
    zq = jnp.zeros((N, qw), _f32)
    n_arr = len(xhs)
    flat = tuple(a.reshape(-1, qw) for a in xhs)
    res = _build_segsum(qw, with_counts, single)(*flat, src3, dst3, zq, z16)
    outs = list(res[:n_arr])
    if with_counts:
        outs.append(res[n_arr])
    return outs


TB = 2000


def _moe_body(sh_refs, cnt_ref, xh_refs, Wg_ref, bg_ref, W_ref, b_ref,
              Wr_ref, Wl2_ref, outh_refs, y2_ref, gs_ref):
    i = pl.program_id(0)
    s = jnp.concatenate([r[...] for r in sh_refs], axis=1)
    cnt = jnp.maximum(cnt_ref[...][:, 0:1], 1.0)
    h = s / cnt
    logits = jnp.dot(h, Wg_ref[...], preferred_element_type=_f32) + bg_ref[...]
    m = jnp.max(logits, axis=1, keepdims=True)
    eg = jnp.exp(logits - m)
    g = eg / jnp.sum(eg, axis=1, keepdims=True)
    gm = jnp.mean(g, axis=1, keepdims=True)
    stds = jnp.sqrt(jnp.sum((g - gm) ** 2, axis=1, keepdims=True) / (NE - 1))

    @pl.when(i == 0)
    def _():
        gs_ref[...] = jnp.zeros((1, 1), _f32)

    gs_ref[...] = gs_ref[...] + jnp.sum(stds).reshape(1, 1)

    x = jnp.concatenate([r[...] for r in xh_refs], axis=1)
    acc = jnp.dot(x, Wr_ref[...], preferred_element_type=_f32)
    found = jnp.zeros((TB, 1), jnp.bool_)
    for e in range(NE):
        is_e = jnp.logical_and(logits[:, e:e + 1] == m,
                               jnp.logical_not(found))
        found = jnp.logical_or(found, is_e)
        mask = is_e.astype(_f32)
        acc += jnp.dot(h * mask, W_ref[e],
                       preferred_element_type=_f32) + mask * b_ref[e:e + 1, :]
    xn = jnp.maximum(acc, 0.0)
    outh_refs[0][...] = xn[:, :128]
    outh_refs[1][...] = xn[:, 128:]
    if y2_ref is not None:
        y2_ref[...] = jnp.dot(xn, Wl2_ref[...], preferred_element_type=_f32)


@functools.lru_cache(maxsize=None)
def _make_moe_layer(with_y2: bool, interpret: bool = False):
    def body(*refs):
        sh = refs[0:2]
        cnt = refs[2]
        xh = refs[3:5]
        Wg, bg, W, b, Wr = refs[5:10]
        k = 10
        Wl2 = refs[k] if with_y2 else None
        k += 1 if with_y2 else 0
        outh = refs[k:k + 2]
        k += 2
        y2 = refs[k] if with_y2 else None
        k += 1 if with_y2 else 0
        gs = refs[k]
        _moe_body(sh, cnt, xh, Wg, bg, W, b, Wr, Wl2, outh, y2, gs)

    grid = (N // TB,)
    tok = lambda w: pl.BlockSpec((TB, w), lambda i: (i, 0))
    full = lambda *shape: pl.BlockSpec(shape, lambda i: tuple(0 for _ in shape))
    in_specs = [tok(128)] * 2 + [tok(16)] + [tok(128)] * 2 + [
        full(DH, NE), full(1, NE), full(NE, DH, DH), full(NE, DH),
        full(DH, DH)]
    out_shapes = [jax.ShapeDtypeStruct((N, 128), _f32) for _ in range(2)]
    out_specs = [tok(128)] * 2
    if with_y2:
        in_specs.append(full(DH, DOUT))
        out_shapes.append(jax.ShapeDtypeStruct((N, DOUT), _f32))
        out_specs.append(tok(DOUT))
    out_shapes.append(jax.ShapeDtypeStruct((1, 1), _f32))
    out_specs.append(pl.BlockSpec((1, 1), lambda i: (0, 0)))
    return pl.pallas_call(
        body, grid=grid, in_specs=in_specs, out_specs=out_specs,
        out_shape=tuple(out_shapes), interpret=interpret)


def _final_body(s2_ref, cnt_ref, xa_ref, xb_ref, Wr2_ref, bl2_ref, out_ref):
    cnt = jnp.maximum(cnt_ref[...][:, 0:1], 1.0)
    x2 = jnp.concatenate([xa_ref[...], xb_ref[...]], axis=1)
    out_ref[...] = (s2_ref[...] / cnt + bl2_ref[...]
                    + jnp.dot(x2, Wr2_ref[...], preferred_element_type=_f32))


@functools.lru_cache(maxsize=None)
def _make_final(interpret: bool = False):
    return pl.pallas_call(
        _final_body, grid=(N // TB,),
        in_specs=[pl.BlockSpec((TB, 128), lambda i: (i, 0)),
                  pl.BlockSpec((TB, 16), lambda i: (i, 0)),
                  pl.BlockSpec((TB, 128), lambda i: (i, 0)),
                  pl.BlockSpec((TB, 128), lambda i: (i, 0)),
                  pl.BlockSpec((DH, DOUT), lambda i: (0, 0)),
                  pl.BlockSpec((1, DOUT), lambda i: (0, 0))],
        out_specs=pl.BlockSpec((TB, DOUT), lambda i: (i, 0)),
        out_shape=jax.ShapeDtypeStruct((N, DOUT), _f32),
        interpret=interpret)


def kernel(x, edge_index, Wg0, bg0, W0, b0, Wr0, Wg1, bg1, W1, b1, Wr1,
           Wl2, bl2, Wr2):
    xh = (x[:, :128], x[:, 128:])
    src3 = edge_index[0].reshape(NTILES, CHUNKS_PER_TILE, CHUNK)
    dst3 = edge_index[1].reshape(NTILES, CHUNKS_PER_TILE, CHUNK)
    z16 = jnp.zeros((N, 16), _f32)

    *s0h, cnt16 = _segsum(xh, src3, dst3, 64, True, False, z16)
    r0 = _make_moe_layer(False)(*s0h, cnt16, *xh, Wg0, bg0.reshape(1, NE),
                                W0, b0, Wr0)
    x1h, gs0 = r0[:2], r0[2]
    s1h = _segsum(x1h, src3, dst3, 64, False, False, z16)
    r1 = _make_moe_layer(True)(*s1h, cnt16, *x1h, Wg1, bg1.reshape(1, NE),
                               W1, b1, Wr1, Wl2)
    x2h, y2, gs1 = r1[:2], r1[2], r1[3]
    (s2,) = _segsum((y2,), src3, dst3, 32, False, True, z16)
    out = _make_final()(s2, cnt16, *x2h, Wr2, bl2.reshape(1, DOUT))
    gstd = (gs0[0, 0] + gs1[0, 0]) / (2.0 * N)
    return out, gstd
